# baseline (speedup 1.0000x reference)
import functools

import jax
import jax.numpy as jnp
from jax import lax
from jax.experimental import pallas as pl
from jax.experimental.pallas import tpu as pltpu
from jax.experimental.pallas import tpu_sc as plsc

VOCAB = 1000000
EMBED_DIM = 64
BATCH = 4096
SEQ_LEN = 200

NUM_CORES = 2
NUM_SUBCORES = 16
NW = NUM_CORES * NUM_SUBCORES
TOTAL = BATCH * SEQ_LEN
B_PER_W = TOTAL // NW
CHUNK = 128
NCHUNK = B_PER_W // CHUNK
GBUF = 4
GROUP_ROWS = GBUF * CHUNK
NGROUP = NCHUNK // GBUF


@functools.partial(jax.jit, static_argnums=())
def _embed(idx3, table):
    mesh = plsc.VectorSubcoreMesh(
        core_axis_name="c", subcore_axis_name="s",
        num_cores=NUM_CORES, num_subcores=NUM_SUBCORES)

    @functools.partial(
        pl.kernel,
        mesh=mesh,
        out_type=jax.ShapeDtypeStruct((TOTAL // CHUNK, CHUNK, EMBED_DIM),
                                      jnp.float32),
        scratch_types=[
            pltpu.VMEM((NCHUNK, CHUNK), jnp.int32),
            pltpu.VMEM((2, GBUF, CHUNK, EMBED_DIM), jnp.float32),
            pltpu.SemaphoreType.DMA((2,)),
            pltpu.SemaphoreType.DMA((2,)),
        ],
        compiler_params=pltpu.CompilerParams(use_tc_tiling_on_sc=False),
    )
    def emb_kernel(idx_hbm, table_hbm, out_hbm, idx_v, rows_v, gsem, wsem):
        wid = lax.axis_index("s") * NUM_CORES + lax.axis_index("c")
        cbase = wid * NCHUNK
        pltpu.sync_copy(idx_hbm.at[wid], idx_v)

        def group_gather_start(g, h):
            for b in range(GBUF):
                pltpu.async_copy(table_hbm.at[idx_v.at[g * GBUF + b]],
                                 rows_v.at[h, b], gsem.at[h])

        def group_gather_wait(g, h):
            for b in range(GBUF):
                pltpu.make_async_copy(table_hbm.at[idx_v.at[g * GBUF + b]],
                                      rows_v.at[h, b], gsem.at[h]).wait()

        def group_out(g):
            return out_hbm.at[pl.ds(cbase + g * GBUF, GBUF)]

        def write_start(g, h):
            pltpu.async_copy(rows_v.at[h], group_out(g), wsem.at[h])

        def write_wait(g, h):
            pltpu.make_async_copy(rows_v.at[h], group_out(g),
                                  wsem.at[h]).wait()

        group_gather_start(0, 0)
        npairs = NGROUP // 2

        def body(p, carry):
            g0 = 2 * p
            g1 = g0 + 1
            group_gather_wait(g0, 0)

            @pl.when(p >= 1)
            def _():
                write_wait(g0 - 1, 1)

            group_gather_start(g1, 1)
            write_start(g0, 0)
            group_gather_wait(g1, 1)
            write_wait(g0, 0)

            @pl.when(p < npairs - 1)
            def _():
                group_gather_start(g0 + 2, 0)

            write_start(g1, 1)
            return carry

        lax.fori_loop(0, npairs, body, 0)
        write_wait(NGROUP - 1, 1)

    return emb_kernel(idx3, table)


def kernel(text, table):
    idx3 = text.reshape(NW, NCHUNK, CHUNK)
    out = _embed(idx3, table)
    return out.reshape(BATCH, SEQ_LEN, EMBED_DIM)
